# baseline (speedup 1.0000x reference)
import functools

import jax
import jax.numpy as jnp
from jax import lax
from jax.experimental import pallas as pl
from jax.experimental.pallas import tpu as pltpu
from jax.experimental.pallas import tpu_sc as plsc

B = 64
N = 1024
D = 768
HALF = D // 2
NC = 2
NS = 16
NW = NC * NS
T = B * N
TPW = T // NW
C = 64
STEPS = TPW // C
LPR = HALF // 16


def _sc_pe_add(x2, rows_f, cols_f, pr, pc):
    mesh = plsc.VectorSubcoreMesh(core_axis_name="c", subcore_axis_name="s")

    @functools.partial(
        pl.kernel,
        mesh=mesh,
        out_type=jax.ShapeDtypeStruct((2 * T * HALF,), jnp.float32),
        scratch_types=[
            pltpu.VMEM((C,), jnp.int32),
            pltpu.VMEM((C,), jnp.int32),
            pltpu.VMEM((C, HALF), jnp.float32),
            pltpu.VMEM((C, HALF), jnp.float32),
            pltpu.VMEM((2 * C * HALF,), jnp.float32),
            pltpu.SemaphoreType.DMA,
            pltpu.SemaphoreType.DMA,
        ],
    )
    def k(x_hbm, rows_hbm, cols_hbm, pr_hbm, pc_hbm, out_hbm,
          idxr_v, idxc_v, prb, pcb, xb, semr, semc):
        wid = lax.axis_index("s") * NC + lax.axis_index("c")

        def step(s, carry):
            base = wid * TPW + s * C
            pltpu.sync_copy(rows_hbm.at[pl.ds(base, C)], idxr_v)
            pltpu.sync_copy(cols_hbm.at[pl.ds(base, C)], idxc_v)
            cr = pltpu.async_copy(pr_hbm.at[idxr_v], prb, semr)
            cc = pltpu.async_copy(pc_hbm.at[idxc_v], pcb, semc)
            pltpu.sync_copy(x_hbm.at[pl.ds(base * D, C * D)], xb)
            cr.wait()
            cc.wait()

            def tok(i, carry2):
                xoff = i * D
                for j in range(LPR):
                    plsc.addupdate(xb.at[pl.ds(xoff + j * 16, 16)],
                                   prb[i, pl.ds(j * 16, 16)])
                    plsc.addupdate(xb.at[pl.ds(xoff + HALF + j * 16, 16)],
                                   pcb[i, pl.ds(j * 16, 16)])
                return carry2

            lax.fori_loop(0, C, tok, 0)
            pltpu.sync_copy(xb, out_hbm.at[pl.ds(base * D, C * D)])
            return carry

        lax.fori_loop(0, STEPS, step, 0)

    return k(x2, rows_f, cols_f, pr, pc)


def kernel(x, rows, cols, pe_row, pe_col):
    x2 = x.reshape(2 * T * HALF)
    rows_f = rows.reshape(T).astype(jnp.int32)
    cols_f = cols.reshape(T).astype(jnp.int32)
    pr = 0.1 * pe_row
    pc = 0.1 * pe_col
    out2 = _sc_pe_add(x2, rows_f, cols_f, pr, pc)
    return out2.reshape(B, N, D)


# speedup vs baseline: 1.2346x; 1.2346x over previous
import functools

import jax
import jax.numpy as jnp
from jax import lax
from jax.experimental import pallas as pl
from jax.experimental.pallas import tpu as pltpu
from jax.experimental.pallas import tpu_sc as plsc

B = 64
N = 1024
D = 768
HALF = D // 2
NC = 2
NS = 16
NW = NC * NS
T = B * N
TPW = T // NW
C = 32
STEPS = TPW // C
G = STEPS // 2
LPR = HALF // 16


def _sc_pe_add(x2, rows_f, cols_f, pr, pc):
    mesh = plsc.VectorSubcoreMesh(core_axis_name="c", subcore_axis_name="s")

    @functools.partial(
        pl.kernel,
        mesh=mesh,
        out_type=jax.ShapeDtypeStruct((2 * T * HALF,), jnp.float32),
        scratch_types=[
            pltpu.VMEM((TPW,), jnp.int32),
            pltpu.VMEM((TPW,), jnp.int32),
            pltpu.VMEM((C, HALF), jnp.float32),
            pltpu.VMEM((C, HALF), jnp.float32),
            pltpu.VMEM((C, HALF), jnp.float32),
            pltpu.VMEM((C, HALF), jnp.float32),
            pltpu.VMEM((C * D,), jnp.float32),
            pltpu.VMEM((C * D,), jnp.float32),
        ] + [pltpu.SemaphoreType.DMA] * 8,
    )
    def k(x_hbm, rows_hbm, cols_hbm, pr_hbm, pc_hbm, out_hbm,
          idxr_a, idxc_a, prb0, prb1, pcb0, pcb1, xb0, xb1,
          sgr0, sgr1, sgc0, sgc1, sx0, sx1, so0, so1):
        wid = lax.axis_index("s") * NC + lax.axis_index("c")
        prb, pcb, xb = (prb0, prb1), (pcb0, pcb1), (xb0, xb1)
        sgr, sgc, sx, so = (sgr0, sgr1), (sgc0, sgc1), (sx0, sx1), (so0, so1)

        pltpu.sync_copy(rows_hbm.at[pl.ds(wid * TPW, TPW)], idxr_a)
        pltpu.sync_copy(cols_hbm.at[pl.ds(wid * TPW, TPW)], idxc_a)

        def issue_in(s, b):
            loc = s * C
            pltpu.async_copy(pr_hbm.at[idxr_a.at[pl.ds(loc, C)]], prb[b], sgr[b])
            pltpu.async_copy(pc_hbm.at[idxc_a.at[pl.ds(loc, C)]], pcb[b], sgc[b])
            pltpu.async_copy(x_hbm.at[pl.ds((wid * TPW + s * C) * D, C * D)],
                             xb[b], sx[b])

        def wait_in(s, b):
            loc = s * C
            pltpu.make_async_copy(
                pr_hbm.at[idxr_a.at[pl.ds(loc, C)]], prb[b], sgr[b]).wait()
            pltpu.make_async_copy(
                pc_hbm.at[idxc_a.at[pl.ds(loc, C)]], pcb[b], sgc[b]).wait()
            pltpu.make_async_copy(
                x_hbm.at[pl.ds((wid * TPW + s * C) * D, C * D)],
                xb[b], sx[b]).wait()

        def issue_out(s, b):
            pltpu.async_copy(xb[b],
                             out_hbm.at[pl.ds((wid * TPW + s * C) * D, C * D)],
                             so[b])

        def wait_out(s, b):
            pltpu.make_async_copy(
                xb[b], out_hbm.at[pl.ds((wid * TPW + s * C) * D, C * D)],
                so[b]).wait()

        def compute(b):
            xbb, prbb, pcbb = xb[b], prb[b], pcb[b]

            def tok(i, carry2):
                xoff = i * D
                for j in range(LPR):
                    plsc.addupdate(xbb.at[pl.ds(xoff + j * 16, 16)],
                                   prbb[i, pl.ds(j * 16, 16)])
                    plsc.addupdate(xbb.at[pl.ds(xoff + HALF + j * 16, 16)],
                                   pcbb[i, pl.ds(j * 16, 16)])
                return carry2

            lax.fori_loop(0, C, tok, 0)

        def group(g, first, last):
            for b in (0, 1):
                s = g * 2 + b
                nb = 1 - b
                if not (last and b == 1):
                    if not (first and b == 0):
                        wait_out(s - 1, nb)
                    issue_in(s + 1, nb)
                wait_in(s, b)
                compute(b)
                issue_out(s, b)
            return 0

        issue_in(0, 0)
        group(0, True, False)
        lax.fori_loop(1, G - 1, lambda g, c: group(g, False, False), 0)
        group(G - 1, False, True)
        wait_out(STEPS - 2, 0)
        wait_out(STEPS - 1, 1)

    return k(x2, rows_f, cols_f, pr, pc)


def kernel(x, rows, cols, pe_row, pe_col):
    x2 = x.reshape(2 * T * HALF)
    rows_f = rows.reshape(T).astype(jnp.int32)
    cols_f = cols.reshape(T).astype(jnp.int32)
    pr = 0.1 * pe_row
    pc = 0.1 * pe_col
    out2 = _sc_pe_add(x2, rows_f, cols_f, pr, pc)
    return out2.reshape(B, N, D)


# speedup vs baseline: 2.3492x; 1.9028x over previous
import functools

import jax
import jax.numpy as jnp
from jax import lax
from jax.experimental import pallas as pl
from jax.experimental.pallas import tpu as pltpu
from jax.experimental.pallas import tpu_sc as plsc

B = 64
N = 1024
D = 768
HALF = D // 2
NC = 2
NS = 16
NW = NC * NS
T = B * N
TPW = T // NW
C = 32
STEPS = TPW // C
G = STEPS // 2
LPR = HALF // 16


def _sc_pe_add(x2, rows_f, cols_f, pr, pc):
    mesh = plsc.VectorSubcoreMesh(core_axis_name="c", subcore_axis_name="s")

    @functools.partial(
        pl.kernel,
        mesh=mesh,
        out_type=jax.ShapeDtypeStruct((T, D), jnp.float32),
        scratch_types=[
            pltpu.VMEM((TPW,), jnp.int32),
            pltpu.VMEM((TPW,), jnp.int32),
            pltpu.VMEM((C, HALF), jnp.float32),
            pltpu.VMEM((C, HALF), jnp.float32),
            pltpu.VMEM((C, HALF), jnp.float32),
            pltpu.VMEM((C, HALF), jnp.float32),
            pltpu.VMEM((C, D), jnp.float32),
            pltpu.VMEM((C, D), jnp.float32),
        ] + [pltpu.SemaphoreType.DMA] * 8,
    )
    def k(x_hbm, rows_hbm, cols_hbm, pr_hbm, pc_hbm, out_hbm,
          idxr_a, idxc_a, prb0, prb1, pcb0, pcb1, xb0, xb1,
          sgr0, sgr1, sgc0, sgc1, sx0, sx1, so0, so1):
        wid = lax.axis_index("s") * NC + lax.axis_index("c")
        prb, pcb, xb = (prb0, prb1), (pcb0, pcb1), (xb0, xb1)
        sgr, sgc, sx, so = (sgr0, sgr1), (sgc0, sgc1), (sx0, sx1), (so0, so1)

        pltpu.sync_copy(rows_hbm.at[pl.ds(wid * TPW, TPW)], idxr_a)
        pltpu.sync_copy(cols_hbm.at[pl.ds(wid * TPW, TPW)], idxc_a)

        def issue_in(s, b):
            loc = s * C
            pltpu.async_copy(pr_hbm.at[idxr_a.at[pl.ds(loc, C)]], prb[b], sgr[b])
            pltpu.async_copy(pc_hbm.at[idxc_a.at[pl.ds(loc, C)]], pcb[b], sgc[b])
            pltpu.async_copy(x_hbm.at[pl.ds(wid * TPW + s * C, C)], xb[b], sx[b])

        def wait_in(s, b):
            loc = s * C
            pltpu.make_async_copy(
                pr_hbm.at[idxr_a.at[pl.ds(loc, C)]], prb[b], sgr[b]).wait()
            pltpu.make_async_copy(
                pc_hbm.at[idxc_a.at[pl.ds(loc, C)]], pcb[b], sgc[b]).wait()
            pltpu.make_async_copy(
                x_hbm.at[pl.ds(wid * TPW + s * C, C)], xb[b], sx[b]).wait()

        def issue_out(s, b):
            pltpu.async_copy(xb[b], out_hbm.at[pl.ds(wid * TPW + s * C, C)],
                             so[b])

        def wait_out(s, b):
            pltpu.make_async_copy(
                xb[b], out_hbm.at[pl.ds(wid * TPW + s * C, C)], so[b]).wait()

        def compute(b):
            xbb, prbb, pcbb = xb[b], prb[b], pcb[b]

            def tok(i, carry2):
                for j in range(LPR):
                    plsc.addupdate(xbb.at[i, pl.ds(j * 16, 16)],
                                   prbb[i, pl.ds(j * 16, 16)])
                    plsc.addupdate(xbb.at[i, pl.ds(HALF + j * 16, 16)],
                                   pcbb[i, pl.ds(j * 16, 16)])
                return carry2

            lax.fori_loop(0, C, tok, 0)

        def group(g, first, last):
            for b in (0, 1):
                s = g * 2 + b
                nb = 1 - b
                if not (last and b == 1):
                    if not (first and b == 0):
                        wait_out(s - 1, nb)
                    issue_in(s + 1, nb)
                wait_in(s, b)
                compute(b)
                issue_out(s, b)
            return 0

        issue_in(0, 0)
        group(0, True, False)
        lax.fori_loop(1, G - 1, lambda g, c: group(g, False, False), 0)
        group(G - 1, False, True)
        wait_out(STEPS - 2, 0)
        wait_out(STEPS - 1, 1)

    return k(x2, rows_f, cols_f, pr, pc)


def kernel(x, rows, cols, pe_row, pe_col):
    x2 = x.reshape(T, D)
    rows_f = rows.reshape(T).astype(jnp.int32)
    cols_f = cols.reshape(T).astype(jnp.int32)
    pr = 0.1 * pe_row
    pc = 0.1 * pe_col
    out2 = _sc_pe_add(x2, rows_f, cols_f, pr, pc)
    return out2.reshape(B, N, D)
